# baseline (speedup 1.0000x reference)
import functools

import jax
import jax.numpy as jnp
from jax import lax
from jax.experimental import pallas as pl
from jax.experimental.pallas import tpu as pltpu
from jax.experimental.pallas import tpu_sc as plsc

STEP = 512
NBUF = 2


@functools.partial(jax.jit, static_argnames=("n_workers",))
def _embed_sc(x_flat, table, n_workers):
    b_total = x_flat.shape[0]
    d = table.shape[1]
    per_w = b_total // n_workers
    n_steps = per_w // STEP
    step_rows = STEP
    assert per_w % STEP == 0
    assert n_steps % NBUF == 0 and n_steps >= 2 * NBUF

    mesh = plsc.VectorSubcoreMesh(core_axis_name="c", subcore_axis_name="s")

    @functools.partial(
        pl.kernel,
        out_type=jax.ShapeDtypeStruct((b_total, d), jnp.float32),
        mesh=mesh,
        scratch_types=[
            pltpu.VMEM((per_w,), jnp.int32),
            pltpu.VMEM((NBUF, step_rows, d), jnp.float32),
        ]
        + [pltpu.SemaphoreType.DMA] * (2 * NBUF),
        compiler_params=pltpu.CompilerParams(use_tc_tiling_on_sc=False),
    )
    def k(idx_hbm, table_hbm, out_hbm, idx_v, rows_v, *sems):
        gsem = sems[:NBUF]
        wsem = sems[NBUF:]
        wid = lax.axis_index("s") * 2 + lax.axis_index("c")
        row_base = wid * per_w

        pltpu.sync_copy(idx_hbm.at[pl.ds(row_base, per_w)], idx_v)

        def start_gather(s, b):
            pltpu.make_async_copy(
                table_hbm.at[idx_v.at[pl.ds(s * STEP, STEP)]], rows_v.at[b], gsem[b]
            ).start()

        def wait_gather(b):
            pltpu.make_async_copy(
                table_hbm.at[idx_v.at[pl.ds(0, STEP)]], rows_v.at[b], gsem[b]
            ).wait()

        def start_write(s, b):
            pltpu.make_async_copy(
                rows_v.at[b],
                out_hbm.at[pl.ds(row_base + s * step_rows, step_rows)],
                wsem[b],
            ).start()

        def wait_write(b):
            pltpu.make_async_copy(
                rows_v.at[b], out_hbm.at[pl.ds(row_base, step_rows)], wsem[b]
            ).wait()

        for b in range(NBUF):
            start_gather(b, b)

        def block(g, carry):
            for b in range(NBUF):
                wait_gather(b)
                start_write(g + b, b)
            for b in range(NBUF):
                wait_write(b)
                start_gather(g + b + NBUF, b)
            return carry

        lax.fori_loop(0, (n_steps - NBUF) // NBUF, lambda i, c: block(i * NBUF, c), 0)

        g_last = n_steps - NBUF
        for b in range(NBUF):
            wait_gather(b)
            start_write(g_last + b, b)
        for b in range(NBUF):
            wait_write(b)

    return k(x_flat, table)


def kernel(x, table):
    orig_shape = x.shape
    x_flat = x.reshape(-1).astype(jnp.int32)
    out = _embed_sc(x_flat, table, 32)
    return out.reshape(*orig_shape, table.shape[1])
